# baseline (speedup 1.0000x reference)
import functools

import jax
import jax.numpy as jnp
from jax import lax
from jax.experimental import pallas as pl
from jax.experimental.pallas import tpu as pltpu
from jax.experimental.pallas import tpu_sc as plsc

B, H, D = 16384, 50, 16
V = 1000000
N = B * H
NC, NS = 2, 16
NW = NC * NS

_mesh = plsc.VectorSubcoreMesh(
    core_axis_name="c", subcore_axis_name="s", num_cores=NC, num_subcores=NS
)

FULL_COLS = V // 128
TAIL = V - FULL_COLS * 128
BASE_COLS = FULL_COLS // NW
EXTRA = FULL_COLS - BASE_COLS * NW
BLK = 2
STEPS_A = BASE_COLS // BLK
LANES = BLK * 128


@functools.partial(
    pl.kernel,
    out_type=jax.ShapeDtypeStruct((V // 8, 128), jnp.float32),
    mesh=_mesh,
    compiler_params=pltpu.CompilerParams(
        use_tc_tiling_on_sc=True, needs_layout_passes=False
    ),
    scratch_types=[
        pltpu.VMEM((2, 16, LANES), jnp.float32),
        pltpu.VMEM((2, 16, LANES), jnp.float32),
        pltpu.VMEM((2, BLK * 16, 128), jnp.float32),
        pltpu.VMEM((8, 128), jnp.float32),
        pltpu.SemaphoreType.DMA,
        pltpu.SemaphoreType.DMA,
        pltpu.SemaphoreType.DMA,
        pltpu.SemaphoreType.DMA,
    ],
)
def _sum_tables(item_hbm, ent_hbm, tail_hbm, s2_hbm, item_v, ent_v, srow_v,
                tail_v, si0, si1, so0, so1):
    wid = lax.axis_index("s") * NC + lax.axis_index("c")
    col0 = wid * BASE_COLS
    riota = jax.lax.iota(jnp.int32, 16)
    si = (si0, si1)
    so = (so0, so1)

    def fire_in(s, b):
        c = col0 + s * BLK
        pltpu.async_copy(item_hbm.at[:, pl.ds(c * 128, LANES)],
                         item_v.at[b], si[b])
        pltpu.async_copy(ent_hbm.at[:, pl.ds(c * 128, LANES)],
                         ent_v.at[b], si[b])

    def wait_in(b):
        pltpu.make_async_copy(item_hbm.at[:, pl.ds(0, LANES)],
                              item_v.at[b], si[b]).wait()
        pltpu.make_async_copy(ent_hbm.at[:, pl.ds(0, LANES)],
                              ent_v.at[b], si[b]).wait()

    def compute(b):
        for l in range(LANES):
            cidx = jnp.full((16,), l, jnp.int32)
            col = (plsc.load_gather(item_v.at[b], [riota, cidx])
                   + plsc.load_gather(ent_v.at[b], [riota, cidx]))
            plsc.store_scatter(
                srow_v.at[b],
                [jnp.full((16,), l // 8, jnp.int32), (l % 8) * 16 + riota],
                col,
            )

    def fire_out(s, b):
        c = col0 + s * BLK
        pltpu.async_copy(srow_v.at[b],
                         s2_hbm.at[pl.ds(c * 16, BLK * 16), :], so[b])

    def wait_out(b):
        pltpu.make_async_copy(srow_v.at[b],
                              s2_hbm.at[pl.ds(0, BLK * 16), :], so[b]).wait()

    fire_in(0, 0)

    @pl.loop(0, STEPS_A // 2)
    def _pair(g):
        s0 = 2 * g
        wait_in(0)
        fire_in(s0 + 1, 1)

        @pl.when(g > 0)
        def _():
            wait_out(0)

        compute(0)
        fire_out(s0, 0)

        wait_in(1)

        @pl.when(g < STEPS_A // 2 - 1)
        def _():
            fire_in(s0 + 2, 0)

        @pl.when(g > 0)
        def _():
            wait_out(1)

        compute(1)
        fire_out(s0 + 1, 1)

    wait_out(0)
    wait_out(1)

    @pl.when(wid < EXTRA)
    def _extra():
        c = NW * BASE_COLS + wid
        pltpu.async_copy(item_hbm.at[:, pl.ds(c * 128, 128)],
                         item_v.at[0, :, pl.ds(0, 128)], si[0])
        pltpu.async_copy(ent_hbm.at[:, pl.ds(c * 128, 128)],
                         ent_v.at[0, :, pl.ds(0, 128)], si[0])
        pltpu.make_async_copy(item_hbm.at[:, pl.ds(0, 128)],
                              item_v.at[0, :, pl.ds(0, 128)], si[0]).wait()
        pltpu.make_async_copy(ent_hbm.at[:, pl.ds(0, 128)],
                              ent_v.at[0, :, pl.ds(0, 128)], si[0]).wait()
        @pl.loop(0, 128)
        def _l(l):
            cidx = jnp.full((16,), 1, jnp.int32) * l
            col = (plsc.load_gather(item_v.at[0], [riota, cidx])
                   + plsc.load_gather(ent_v.at[0], [riota, cidx]))
            plsc.store_scatter(
                srow_v.at[0],
                [jnp.full((16,), 1, jnp.int32) * (l // 8),
                 (l % 8) * 16 + riota],
                col,
            )
        pltpu.sync_copy(srow_v.at[0, pl.ds(0, 16), :],
                        s2_hbm.at[pl.ds(c * 16, 16), :])

    @pl.when(wid == NW - 1)
    def _tail():
        pltpu.sync_copy(tail_hbm, tail_v)
        pltpu.sync_copy(tail_v, s2_hbm.at[pl.ds(FULL_COLS * 16, 8), :])


ROWS_PER_W = N // NW
K = 8
C = K * 128
STEPS = ROWS_PER_W // C
IDX_ROWS_PER_W = ROWS_PER_W // 128


@functools.partial(
    pl.kernel,
    out_type=jax.ShapeDtypeStruct((N, D), jnp.float32),
    mesh=_mesh,
    compiler_params=pltpu.CompilerParams(use_tc_tiling_on_sc=False),
    scratch_types=[
        pltpu.VMEM((K, 128), jnp.int32),
        pltpu.VMEM((C, D), jnp.float32),
        pltpu.SemaphoreType.DMA,
    ],
)
def _encode(table_hbm, idx_hbm, out_hbm, idx_v, rows_v, sem):
    wid = lax.axis_index("s") * NC + lax.axis_index("c")
    idx_row0 = wid * IDX_ROWS_PER_W
    out_row0 = wid * ROWS_PER_W

    @pl.loop(0, STEPS)
    def _step(s):
        pltpu.sync_copy(idx_hbm.at[pl.ds(idx_row0 + s * K, K)], idx_v)
        cps = [
            pltpu.async_copy(
                table_hbm.at[idx_v.at[j]], rows_v.at[pl.ds(j * 128, 128)], sem
            )
            for j in range(K)
        ]
        for cp in cps:
            cp.wait()
        pltpu.sync_copy(rows_v, out_hbm.at[pl.ds(out_row0 + s * C, C)])


def kernel(batch_data, item_table, ent_table):
    idx = batch_data.reshape(N // 128, 128).astype(jnp.int32)
    tail = (item_table[FULL_COLS * 128:] + ent_table[FULL_COLS * 128:])
    tail2d = tail.reshape(8, 128)
    s2 = _sum_tables(item_table.T, ent_table.T, tail2d)
    summed = s2.reshape(V, D)
    out = _encode(summed, idx)
    return out.reshape(B, H, D)


# speedup vs baseline: 1.2841x; 1.2841x over previous
import functools

import jax
import jax.numpy as jnp
from jax import lax
from jax.experimental import pallas as pl
from jax.experimental.pallas import tpu as pltpu
from jax.experimental.pallas import tpu_sc as plsc

B, H, D = 16384, 50, 16
V = 1000000
N = B * H
NC, NS = 2, 16
NW = NC * NS

_mesh = plsc.VectorSubcoreMesh(
    core_axis_name="c", subcore_axis_name="s", num_cores=NC, num_subcores=NS
)

ROWS_PER_W = N // NW
K = 8
C = K * 128
STEPS = ROWS_PER_W // C
IDX_ROWS_PER_W = ROWS_PER_W // 128


@functools.partial(
    pl.kernel,
    out_type=jax.ShapeDtypeStruct((N, D), jnp.float32),
    mesh=_mesh,
    compiler_params=pltpu.CompilerParams(use_tc_tiling_on_sc=False),
    scratch_types=[
        pltpu.VMEM((K, 128), jnp.int32),
        pltpu.VMEM((C, D), jnp.float32),
        pltpu.SemaphoreType.DMA,
    ],
)
def _encode(table_hbm, idx_hbm, out_hbm, idx_v, rows_v, sem):
    wid = lax.axis_index("s") * NC + lax.axis_index("c")
    idx_row0 = wid * IDX_ROWS_PER_W
    out_row0 = wid * ROWS_PER_W

    @pl.loop(0, STEPS)
    def _step(s):
        pltpu.sync_copy(idx_hbm.at[pl.ds(idx_row0 + s * K, K)], idx_v)
        cps = [
            pltpu.async_copy(
                table_hbm.at[idx_v.at[j]], rows_v.at[pl.ds(j * 128, 128)], sem
            )
            for j in range(K)
        ]
        for cp in cps:
            cp.wait()
        pltpu.sync_copy(rows_v, out_hbm.at[pl.ds(out_row0 + s * C, C)])


def kernel(batch_data, item_table, ent_table):
    idx = batch_data.reshape(N // 128, 128).astype(jnp.int32)
    summed = item_table + ent_table
    out = _encode(summed, idx)
    return out.reshape(B, H, D)


# speedup vs baseline: 2.1287x; 1.6578x over previous
import functools

import jax
import jax.numpy as jnp
from jax import lax
from jax.experimental import pallas as pl
from jax.experimental.pallas import tpu as pltpu
from jax.experimental.pallas import tpu_sc as plsc

B, H, D = 16384, 50, 16
V = 1000000
N = B * H
NC, NS = 2, 16
NW = NC * NS

_mesh = plsc.VectorSubcoreMesh(
    core_axis_name="c", subcore_axis_name="s", num_cores=NC, num_subcores=NS
)

ROWS_PER_W = N // NW
K = 8
C = K * 128
STEPS = ROWS_PER_W // C
IDX_ROWS_PER_W = ROWS_PER_W // 128
BT = B // 128


@functools.partial(
    pl.kernel,
    out_type=jax.ShapeDtypeStruct((N * D,), jnp.float32),
    mesh=_mesh,
    compiler_params=pltpu.CompilerParams(
        use_tc_tiling_on_sc=False, needs_layout_passes=False
    ),
    scratch_types=[
        pltpu.VMEM((K, 128), jnp.int32),
        pltpu.VMEM((C, D), jnp.float32),
        pltpu.VMEM((D // 8, K * 8 * 128), jnp.float32),
        pltpu.SemaphoreType.DMA,
    ],
)
def _encode(table_hbm, idx_hbm, out_hbm, idx_v, rows_v, tblk_v, sem):
    wid = lax.axis_index("s") * NC + lax.axis_index("c")
    row0 = wid * IDX_ROWS_PER_W
    riota = jax.lax.iota(jnp.int32, 16)
    cf = [jnp.full((16,), f, jnp.int32) for f in range(D)]

    @pl.loop(0, STEPS)
    def _step(s):
        r0 = row0 + s * K
        h = r0 // 128
        c1 = r0 % 128
        pltpu.sync_copy(idx_hbm.at[pl.ds(r0, K)], idx_v)
        cps = [
            pltpu.async_copy(
                table_hbm.at[idx_v.at[j]], rows_v.at[pl.ds(j * 128, 128)], sem
            )
            for j in range(K)
        ]
        for cp in cps:
            cp.wait()

        @pl.loop(0, K)
        def _tr(j):
            for bg in range(8):
                ridx = j * 128 + bg * 16 + riota
                for f in range(D):
                    col = plsc.load_gather(rows_v, [ridx, cf[f]])
                    off = (j * 8 + (f % 8)) * 128 + bg * 16
                    tblk_v.at[f // 8][pl.ds(off, 16)] = col

        base0 = ((h * 2 + 0) * 128 + c1) * 1024
        base1 = ((h * 2 + 1) * 128 + c1) * 1024
        pltpu.sync_copy(tblk_v.at[0], out_hbm.at[pl.ds(base0, K * 1024)])
        pltpu.sync_copy(tblk_v.at[1], out_hbm.at[pl.ds(base1, K * 1024)])


def kernel(batch_data, item_table, ent_table):
    idx = batch_data.T.reshape(N // 128, 128).astype(jnp.int32)
    summed = item_table + ent_table
    flat = _encode(summed, idx)
    out5 = flat.reshape(H, D // 8, BT, 8, 128)
    return out5.transpose(2, 4, 0, 1, 3).reshape(B, H, D)


# speedup vs baseline: 2.1296x; 1.0004x over previous
import functools

import jax
import jax.numpy as jnp
from jax import lax
from jax.experimental import pallas as pl
from jax.experimental.pallas import tpu as pltpu
from jax.experimental.pallas import tpu_sc as plsc

B, H, D = 16384, 50, 16
V = 1000000
N = B * H
NC, NS = 2, 16
NW = NC * NS

_mesh = plsc.VectorSubcoreMesh(
    core_axis_name="c", subcore_axis_name="s", num_cores=NC, num_subcores=NS
)

ROWS_PER_W = N // NW
K = 8
C = K * 128
STEPS = ROWS_PER_W // C
IDX_ROWS_PER_W = ROWS_PER_W // 128
BT = B // 128


@functools.partial(
    pl.kernel,
    out_type=jax.ShapeDtypeStruct((N * D,), jnp.float32),
    mesh=_mesh,
    compiler_params=pltpu.CompilerParams(
        use_tc_tiling_on_sc=False, needs_layout_passes=False
    ),
    scratch_types=[
        pltpu.VMEM((K, 128), jnp.int32),
        pltpu.VMEM((C, D), jnp.float32),
        pltpu.VMEM((D // 8, K * 8 * 128), jnp.float32),
        pltpu.SemaphoreType.DMA,
    ],
)
def _encode(table_hbm, idx_hbm, out_hbm, idx_v, rows_v, tblk_v, sem):
    wid = lax.axis_index("s") * NC + lax.axis_index("c")
    row0 = wid * IDX_ROWS_PER_W
    riota = jax.lax.iota(jnp.int32, 16)
    cf = [jnp.full((16,), f, jnp.int32) for f in range(D)]

    @pl.loop(0, STEPS)
    def _step(s):
        r0 = row0 + s * K
        h = r0 // 128
        c1 = r0 % 128
        pltpu.sync_copy(idx_hbm.at[pl.ds(r0, K)], idx_v)
        cps = [
            pltpu.async_copy(
                table_hbm.at[idx_v.at[j]], rows_v.at[pl.ds(j * 128, 128)], sem
            )
            for j in range(K)
        ]
        for cp in cps:
            cp.wait()

        @pl.loop(0, K)
        def _tr(j):
            for bg in range(8):
                ridx = j * 128 + bg * 16 + riota
                for f in range(D):
                    col = plsc.load_gather(rows_v, [ridx, cf[f]])
                    off = (j * 8 + (f % 8)) * 128 + bg * 16
                    plsc.store_scatter(tblk_v.at[f // 8], [off + riota], col)

        base0 = ((h * 2 + 0) * 128 + c1) * 1024
        base1 = ((h * 2 + 1) * 128 + c1) * 1024
        pltpu.sync_copy(tblk_v.at[0], out_hbm.at[pl.ds(base0, K * 1024)])
        pltpu.sync_copy(tblk_v.at[1], out_hbm.at[pl.ds(base1, K * 1024)])


def kernel(batch_data, item_table, ent_table):
    idx = batch_data.T.reshape(N // 128, 128).astype(jnp.int32)
    summed = item_table + ent_table
    flat = _encode(summed, idx)
    out5 = flat.reshape(H, D // 8, BT, 8, 128)
    return out5.transpose(2, 4, 0, 1, 3).reshape(B, H, D)
